# baseline (speedup 1.0000x reference)
import functools

import jax
import jax.numpy as jnp
import numpy as np
from jax import lax
from jax.experimental import pallas as pl
from jax.experimental.pallas import tpu as pltpu
from jax.experimental.pallas import tpu_sc as plsc

N = 50000
NP = 50176
BN = 3136
RPT = NP // 16
DUMMY = 50000
HID = 128
HEADS = 8
DH = 16
OUT_DIM = 32
E = 400000
C = 512
CH = 50
EP = 16 * CH * C
EIP = EP + 2 * C
NODE_TYPES = ("Hash", "Address")
EDGE_TYPES = (("Hash", "h2a", "Address"), ("Address", "a2h", "Hash"))
SRC_OF = {"Hash": "h2a", "Address": "a2h"}


def _blockdiag(rel):
    return jax.scipy.linalg.block_diag(*[rel[h] for h in range(HEADS)])


def _lin_body(x_ref, w_ref, b_ref, o_ref, *, relu):
    y = jnp.dot(x_ref[...], w_ref[...], preferred_element_type=jnp.float32)
    y = y + b_ref[...]
    if relu:
        y = jnp.maximum(y, 0.0)
    o_ref[...] = y


def _linear(x, W, b, relu=False):
    n, din = x.shape
    dout = W.shape[1]
    return pl.pallas_call(
        functools.partial(_lin_body, relu=relu),
        grid=(n // BN,),
        in_specs=[
            pl.BlockSpec((BN, din), lambda i: (i, 0)),
            pl.BlockSpec((din, dout), lambda i: (0, 0)),
            pl.BlockSpec((1, dout), lambda i: (0, 0)),
        ],
        out_specs=pl.BlockSpec((BN, dout), lambda i: (i, 0)),
        out_shape=jax.ShapeDtypeStruct((n, dout), jnp.float32),
    )(x, W, b.reshape(1, dout))


def _out_body(agg_ref, h_ref, w_ref, b_ref, sk_ref, o_ref):
    a = agg_ref[...]
    g = 0.5 * a * (1.0 + lax.erf(a * np.float32(1.0 / np.sqrt(2.0))))
    y = jnp.dot(g, w_ref[...], preferred_element_type=jnp.float32) + b_ref[...]
    s = jax.nn.sigmoid(sk_ref[0, 0])
    o_ref[...] = s * y + (1.0 - s) * h_ref[...]


def _out_stage(agg, h, W, b, sk):
    return pl.pallas_call(
        _out_body,
        grid=(NP // BN,),
        in_specs=[
            pl.BlockSpec((BN, HID), lambda i: (i, 0)),
            pl.BlockSpec((BN, HID), lambda i: (i, 0)),
            pl.BlockSpec((HID, HID), lambda i: (0, 0)),
            pl.BlockSpec((1, HID), lambda i: (0, 0)),
            pl.BlockSpec((1, 1), lambda i: (0, 0)),
        ],
        out_specs=pl.BlockSpec((BN, HID), lambda i: (i, 0)),
        out_shape=jax.ShapeDtypeStruct((NP, HID), jnp.float32),
    )(agg, h, W, b.reshape(1, HID), sk.reshape(1, 1))


def _final_body(h_ref, w_ref, b_ref, out_ref, ls_ref):
    o = jnp.dot(h_ref[...], w_ref[...], preferred_element_type=jnp.float32)
    o = o + b_ref[...]
    out_ref[...] = o
    m = jnp.max(o, axis=-1, keepdims=True)
    lse = jnp.log(jnp.sum(jnp.exp(o - m), axis=-1, keepdims=True)) + m
    ls_ref[...] = o - lse


def _final_stage(h, W, b):
    return pl.pallas_call(
        _final_body,
        grid=(NP // BN,),
        in_specs=[
            pl.BlockSpec((BN, HID), lambda i: (i, 0)),
            pl.BlockSpec((HID, OUT_DIM), lambda i: (0, 0)),
            pl.BlockSpec((1, OUT_DIM), lambda i: (0, 0)),
        ],
        out_specs=[
            pl.BlockSpec((BN, OUT_DIM), lambda i: (i, 0)),
            pl.BlockSpec((BN, OUT_DIM), lambda i: (i, 0)),
        ],
        out_shape=[
            jax.ShapeDtypeStruct((NP, OUT_DIM), jnp.float32),
            jax.ShapeDtypeStruct((NP, OUT_DIM), jnp.float32),
        ],
    )(h, W, b.reshape(1, OUT_DIM))


_SC_MESH = plsc.VectorSubcoreMesh(core_axis_name="c", subcore_axis_name="s")

_GDN = lax.GatherDimensionNumbers(
    offset_dims=(), collapsed_slice_dims=(0,), start_index_map=(0,))


def _perm(x, idx16):
    return lax.gather(x, idx16[:, None], _GDN, (1,),
                      mode=lax.GatherScatterMode.PROMISE_IN_BOUNDS)


def _edge_body(qtab, kvtab, si2d, di2d, agg_out,
               si2A, di2A, si2B, di2B, iqA, ikA, iqB, ikB,
               dsA, dsB, qrA, kvrA, qrB, kvrB, msg, eb,
               agg_sh, den_sh, semGA, semGB, semIA, semIB, semS):
    c = lax.axis_index("c")
    s = lax.axis_index("s")
    base_row = s * RPT
    iota16 = lax.iota(jnp.int32, 16)
    zrow = jnp.zeros((16,), jnp.float32)
    bufs = ((si2A, di2A, iqA, ikA, qrA, kvrA, semGA, semIA, dsA),
            (si2B, di2B, iqB, ikB, qrB, kvrB, semGB, semIB, dsB))

    def _idx_load(n, which):
        si2, di2 = bufs[which][0], bufs[which][1]
        semI = bufs[which][7]
        row4 = (s * CH + n) * 4
        a = pltpu.async_copy(si2d.at[pl.ds(row4, 4)], si2, semI)
        b = pltpu.async_copy(di2d.at[pl.ds(row4, 4)], di2, semI)
        return a, b

    def _idx_wait(which):
        si2, di2 = bufs[which][0], bufs[which][1]
        semI = bufs[which][7]
        pltpu.make_async_copy(si2d.at[pl.ds(0, 4)], si2, semI).wait()
        pltpu.make_async_copy(di2d.at[pl.ds(0, 4)], di2, semI).wait()

    def _idx_compute(h, which):
        si2, di2, iq, ik = bufs[which][:4]
        dsc = bufs[which][8]

        def _ix(j, _):
            for t in range(8):
                sl = pl.ds(t * 16, 16)
                sv = si2[j, sl]
                dv = di2[j, sl]
                dsc[j, sl] = dv
                iq[j, sl] = dv * 24 + h
                ik[j, sl] = sv * 12 + (4 + h)
            return 0
        lax.fori_loop(0, 4, _ix, 0)

    def _gather_fire(which):
        iq, ik, qr, kvr = bufs[which][2:6]
        semG = bufs[which][6]
        for j in range(4):
            dsl = pl.ds(j * 128, 128)
            pltpu.async_copy(qtab.at[iq.at[j]], qr.at[dsl], semG)
            pltpu.async_copy(kvtab.at[ik.at[j]], kvr.at[dsl], semG)

    def _gather_wait(which):
        iq, ik, qr, kvr = bufs[which][2:6]
        semG = bufs[which][6]
        for j in range(4):
            dsl = pl.ds(j * 128, 128)
            pltpu.make_async_copy(qtab.at[iq.at[j]], qr.at[dsl], semG).wait()
            pltpu.make_async_copy(kvtab.at[ik.at[j]], kvr.at[dsl], semG).wait()

    def _compute_scatter(which):
        qr, kvr = bufs[which][4], bufs[which][5]
        dsc = bufs[which][8]

        def _grp(g, _):
            base = g * 16
            ps = [qr[base + t, :] * kvr[base + t, pl.ds(0, 16)]
                  for t in range(16)]
            for b in (1, 2, 4, 8):
                mask = (iota16 & b) == 0
                pidx = iota16 ^ b
                nxt = []
                for k in range(len(ps) // 2):
                    u = ps[2 * k]
                    w = ps[2 * k + 1]
                    u = u + _perm(u, pidx)
                    w = w + _perm(w, pidx)
                    nxt.append(jnp.where(mask, u, w))
                ps = nxt
            ev = jnp.exp(ps[0])
            eb[pl.ds(base, 16)] = ev
            for t in range(16):
                msg[base + t, :] = (kvr[base + t, pl.ds(16, 16)]
                                    * jnp.full((16,), ev[t]))
            return 0
        lax.fori_loop(0, 32, _grp, 0)

        for j in range(4):
            ssl = pl.ds(j * 128, 128)
            pltpu.async_copy(msg.at[ssl], agg_sh.at[dsc.at[j]], semS, add=True)
            pltpu.async_copy(eb.at[ssl], den_sh.at[dsc.at[j]], semS, add=True)
        for j in range(4):
            ssl = pl.ds(j * 128, 128)
            pltpu.make_async_copy(msg.at[ssl], agg_sh.at[dsc.at[j]], semS).wait()
            pltpu.make_async_copy(eb.at[ssl], den_sh.at[dsc.at[j]], semS).wait()

    def _head(hh, _):
        h = c * 4 + hh
        def _zi(i, _):
            msg[i, :] = zrow
            return 0
        lax.fori_loop(0, C, _zi, 0)

        def _zid(i, _):
            eb[pl.ds(i * 16, 16)] = zrow
            return 0
        lax.fori_loop(0, C // 16, _zid, 0)
        for w in range(7):
            wsz = 512 if w < 6 else 64
            pltpu.sync_copy(msg.at[pl.ds(0, wsz)],
                            agg_sh.at[pl.ds(base_row + w * 512, wsz)])
            pltpu.sync_copy(eb.at[pl.ds(0, wsz)],
                            den_sh.at[pl.ds(base_row + w * 512, wsz)])
        plsc.subcore_barrier()

        _idx_load(0, 0)
        _idx_wait(0)
        _idx_compute(h, 0)
        _gather_fire(0)
        _idx_load(1, 1)

        def _pair(ii, _):
            _idx_wait(1)
            _idx_compute(h, 1)
            _gather_fire(1)
            _idx_load(2 * ii + 2, 0)
            _gather_wait(0)
            _compute_scatter(0)
            _idx_wait(0)
            _idx_compute(h, 0)
            _gather_fire(0)
            _idx_load(2 * ii + 3, 1)
            _gather_wait(1)
            _compute_scatter(1)
            return 0
        lax.fori_loop(0, CH // 2, _pair, 0)
        _gather_wait(0)
        _idx_wait(1)
        plsc.subcore_barrier()

        for w in range(7):
            wsz = 512 if w < 6 else 64
            r0 = base_row + w * 512
            pltpu.sync_copy(agg_sh.at[pl.ds(r0, wsz)], msg.at[pl.ds(0, wsz)])
            pltpu.sync_copy(den_sh.at[pl.ds(r0, wsz)], eb.at[pl.ds(0, wsz)])

            def _div(j, _):
                r16 = 1.0 / (eb[pl.ds(j * 16, 16)] + 1e-16)
                for t in range(16):
                    r = j * 16 + t
                    msg[r, :] = msg[r, :] * jnp.full((16,), r16[t])
                return 0
            lax.fori_loop(0, wsz // 16, _div, 0)
            pltpu.sync_copy(msg.at[pl.ds(0, wsz)], agg_out.at[pl.ds(r0, wsz), h])
        return 0
    lax.fori_loop(0, 4, _head, 0)


def _edge_pass(cat_dst, cat_src, si2d, di2d):
    f = pl.kernel(
        _edge_body,
        out_type=jax.ShapeDtypeStruct((NP, HEADS, DH), jnp.float32),
        mesh=_SC_MESH,
        compiler_params=pltpu.CompilerParams(use_tc_tiling_on_sc=False),
        scratch_types=[
            pltpu.VMEM((4, 128), jnp.int32),
            pltpu.VMEM((4, 128), jnp.int32),
            pltpu.VMEM((4, 128), jnp.int32),
            pltpu.VMEM((4, 128), jnp.int32),
            pltpu.VMEM((4, 128), jnp.int32),
            pltpu.VMEM((4, 128), jnp.int32),
            pltpu.VMEM((4, 128), jnp.int32),
            pltpu.VMEM((4, 128), jnp.int32),
            pltpu.VMEM((4, 128), jnp.int32),
            pltpu.VMEM((4, 128), jnp.int32),
            pltpu.VMEM((C, 16), jnp.float32),
            pltpu.VMEM((C, 32), jnp.float32),
            pltpu.VMEM((C, 16), jnp.float32),
            pltpu.VMEM((C, 32), jnp.float32),
            pltpu.VMEM((C, 16), jnp.float32),
            pltpu.VMEM((C,), jnp.float32),
            pltpu.VMEM_SHARED((NP, 16), jnp.float32),
            pltpu.VMEM_SHARED((NP,), jnp.float32),
            pltpu.SemaphoreType.DMA,
            pltpu.SemaphoreType.DMA,
            pltpu.SemaphoreType.DMA,
            pltpu.SemaphoreType.DMA,
            pltpu.SemaphoreType.DMA,
        ],
    )
    return f(cat_dst.reshape(NP * 3 * HEADS, DH),
             cat_src.reshape(NP * 3 * HEADS // 2, 2 * DH), si2d, di2d)


def kernel(params, x_Hash, x_Address, ei_h2a, ei_a2h):
    x = {
        "Hash": jnp.pad(x_Hash, ((0, NP - N), (0, 0))),
        "Address": jnp.pad(x_Address, ((0, NP - N), (0, 0))),
    }
    eis = {}
    for name, ei in (("h2a", ei_h2a), ("a2h", ei_a2h)):
        si = jnp.pad(ei[0].astype(jnp.int32), (0, EIP - E))
        di = jnp.pad(ei[1].astype(jnp.int32), (0, EIP - E), constant_values=DUMMY)
        eis[name] = (si.reshape(-1, 128), di.reshape(-1, 128))

    h = {}
    for nt in NODE_TYPES:
        W, b = params["lin_in"][nt]
        h[nt] = _linear(x[nt], W, b, relu=True)

    for layer in params["layers"]:
        cat = {}
        for nt in NODE_TYPES:
            name = SRC_OF[nt]
            a_rel, m_rel, p_rel = layer["rel"][name]
            scale = p_rel / np.float32(np.sqrt(DH))
            A = _blockdiag(a_rel * scale[:, None, None])
            M = _blockdiag(m_rel)
            Wq, bq = layer["q"][nt]
            Wk, bk = layer["k"][nt]
            Wv, bv = layer["v"][nt]
            Wk2 = (Wk @ A).reshape(HID, HEADS, DH)
            Wv2 = (Wv @ M).reshape(HID, HEADS, DH)
            Wkv = jnp.stack([Wk2, Wv2], axis=2).reshape(HID, 2 * HID)
            bk2 = (bk @ A).reshape(HEADS, DH)
            bv2 = (bv @ M).reshape(HEADS, DH)
            bkv = jnp.stack([bk2, bv2], axis=1).reshape(2 * HID)
            Wcat = jnp.concatenate([Wq, Wkv], axis=1)
            bcat = jnp.concatenate([bq, bkv])
            cat[nt] = _linear(h[nt], Wcat, bcat)

        agg = {}
        for (src, name, dst) in EDGE_TYPES:
            si2d, di2d = eis[name]
            agg[dst] = _edge_pass(cat[dst], cat[src], si2d, di2d)

        new_h = {}
        for nt in NODE_TYPES:
            Wa, ba = layer["a"][nt]
            new_h[nt] = _out_stage(agg[nt].reshape(NP, HID), h[nt], Wa, ba,
                                   layer["skip"][nt])
        h = new_h

    W, b = params["lin"]
    out, ls = _final_stage(h["Hash"], W, b)
    return out[:N], ls[:N]


# speedup vs baseline: 1.2205x; 1.2205x over previous
import functools

import jax
import jax.numpy as jnp
import numpy as np
from jax import lax
from jax.experimental import pallas as pl
from jax.experimental.pallas import tpu as pltpu
from jax.experimental.pallas import tpu_sc as plsc

N = 50000
NP = 50176
BN = 3136
RPT = NP // 16
DUMMY = 50000
HID = 128
HEADS = 8
DH = 16
OUT_DIM = 32
E = 400000
C = 512
CH = 50
EP = 16 * CH * C
EIP = EP + 2 * C
NODE_TYPES = ("Hash", "Address")
EDGE_TYPES = (("Hash", "h2a", "Address"), ("Address", "a2h", "Hash"))
SRC_OF = {"Hash": "h2a", "Address": "a2h"}


def _blockdiag(rel):
    return jax.scipy.linalg.block_diag(*[rel[h] for h in range(HEADS)])


def _lin_body(x_ref, w_ref, b_ref, o_ref, *, relu):
    y = jnp.dot(x_ref[...], w_ref[...], preferred_element_type=jnp.float32)
    y = y + b_ref[...]
    if relu:
        y = jnp.maximum(y, 0.0)
    o_ref[...] = y


def _linear(x, W, b, relu=False):
    n, din = x.shape
    dout = W.shape[1]
    return pl.pallas_call(
        functools.partial(_lin_body, relu=relu),
        grid=(n // BN,),
        in_specs=[
            pl.BlockSpec((BN, din), lambda i: (i, 0)),
            pl.BlockSpec((din, dout), lambda i: (0, 0)),
            pl.BlockSpec((1, dout), lambda i: (0, 0)),
        ],
        out_specs=pl.BlockSpec((BN, dout), lambda i: (i, 0)),
        out_shape=jax.ShapeDtypeStruct((n, dout), jnp.float32),
    )(x, W, b.reshape(1, dout))


def _out_body(agg_ref, h_ref, w_ref, b_ref, sk_ref, o_ref):
    a = agg_ref[...]
    g = 0.5 * a * (1.0 + lax.erf(a * np.float32(1.0 / np.sqrt(2.0))))
    y = jnp.dot(g, w_ref[...], preferred_element_type=jnp.float32) + b_ref[...]
    s = jax.nn.sigmoid(sk_ref[0, 0])
    o_ref[...] = s * y + (1.0 - s) * h_ref[...]


def _out_stage(agg, h, W, b, sk):
    return pl.pallas_call(
        _out_body,
        grid=(NP // BN,),
        in_specs=[
            pl.BlockSpec((BN, HID), lambda i: (i, 0)),
            pl.BlockSpec((BN, HID), lambda i: (i, 0)),
            pl.BlockSpec((HID, HID), lambda i: (0, 0)),
            pl.BlockSpec((1, HID), lambda i: (0, 0)),
            pl.BlockSpec((1, 1), lambda i: (0, 0)),
        ],
        out_specs=pl.BlockSpec((BN, HID), lambda i: (i, 0)),
        out_shape=jax.ShapeDtypeStruct((NP, HID), jnp.float32),
    )(agg, h, W, b.reshape(1, HID), sk.reshape(1, 1))


def _final_body(h_ref, w_ref, b_ref, out_ref, ls_ref):
    o = jnp.dot(h_ref[...], w_ref[...], preferred_element_type=jnp.float32)
    o = o + b_ref[...]
    out_ref[...] = o
    m = jnp.max(o, axis=-1, keepdims=True)
    lse = jnp.log(jnp.sum(jnp.exp(o - m), axis=-1, keepdims=True)) + m
    ls_ref[...] = o - lse


def _final_stage(h, W, b):
    return pl.pallas_call(
        _final_body,
        grid=(NP // BN,),
        in_specs=[
            pl.BlockSpec((BN, HID), lambda i: (i, 0)),
            pl.BlockSpec((HID, OUT_DIM), lambda i: (0, 0)),
            pl.BlockSpec((1, OUT_DIM), lambda i: (0, 0)),
        ],
        out_specs=[
            pl.BlockSpec((BN, OUT_DIM), lambda i: (i, 0)),
            pl.BlockSpec((BN, OUT_DIM), lambda i: (i, 0)),
        ],
        out_shape=[
            jax.ShapeDtypeStruct((NP, OUT_DIM), jnp.float32),
            jax.ShapeDtypeStruct((NP, OUT_DIM), jnp.float32),
        ],
    )(h, W, b.reshape(1, OUT_DIM))


_SC_MESH = plsc.VectorSubcoreMesh(core_axis_name="c", subcore_axis_name="s")

_GDN = lax.GatherDimensionNumbers(
    offset_dims=(), collapsed_slice_dims=(0,), start_index_map=(0,))


def _perm(x, idx16):
    return lax.gather(x, idx16[:, None], _GDN, (1,),
                      mode=lax.GatherScatterMode.PROMISE_IN_BOUNDS)


def _edge_body(qtab, ktab, si2d, di2d, agg_out,
               si2A, di2A, si2B, di2B, iqA, ikA, ivA, iqB, ikB, ivB,
               dsA, dsB, qrA, krA, vrA, qrB, krB, vrB, msg, eb,
               agg_sh, den_sh, semGA, semGB, semIA, semIB, semS):
    c = lax.axis_index("c")
    s = lax.axis_index("s")
    base_row = s * RPT
    iota16 = lax.iota(jnp.int32, 16)
    zrow = jnp.zeros((16,), jnp.float32)
    bufs = ((si2A, di2A, iqA, ikA, ivA, qrA, krA, vrA, semGA, semIA, dsA),
            (si2B, di2B, iqB, ikB, ivB, qrB, krB, vrB, semGB, semIB, dsB))

    def _idx_load(n, which):
        si2, di2 = bufs[which][0], bufs[which][1]
        semI = bufs[which][9]
        row4 = (s * CH + n) * 4
        a = pltpu.async_copy(si2d.at[pl.ds(row4, 4)], si2, semI)
        b = pltpu.async_copy(di2d.at[pl.ds(row4, 4)], di2, semI)
        return a, b

    def _idx_wait(which):
        si2, di2 = bufs[which][0], bufs[which][1]
        semI = bufs[which][9]
        pltpu.make_async_copy(si2d.at[pl.ds(0, 4)], si2, semI).wait()
        pltpu.make_async_copy(di2d.at[pl.ds(0, 4)], di2, semI).wait()

    def _idx_compute(h, which):
        si2, di2, iq, ik, iv = bufs[which][:5]
        dsc = bufs[which][10]

        def _ix(j, _):
            for t in range(8):
                sl = pl.ds(t * 16, 16)
                sv = si2[j, sl]
                dv = di2[j, sl]
                dsc[j, sl] = dv
                iq[j, sl] = dv * 24 + h
                kb = sv * 24 + (8 + h)
                ik[j, sl] = kb
                iv[j, sl] = kb + 8
            return 0
        lax.fori_loop(0, 4, _ix, 0)

    def _gather_fire(which):
        iq, ik, iv, qr, kr, vr = bufs[which][2:8]
        semG = bufs[which][8]
        for j in range(4):
            dsl = pl.ds(j * 128, 128)
            pltpu.async_copy(qtab.at[iq.at[j]], qr.at[dsl], semG)
            pltpu.async_copy(ktab.at[ik.at[j]], kr.at[dsl], semG)
            pltpu.async_copy(ktab.at[iv.at[j]], vr.at[dsl], semG)

    def _gather_wait(which):
        iq, ik, iv, qr, kr, vr = bufs[which][2:8]
        semG = bufs[which][8]
        for j in range(4):
            dsl = pl.ds(j * 128, 128)
            pltpu.make_async_copy(qtab.at[iq.at[j]], qr.at[dsl], semG).wait()
            pltpu.make_async_copy(ktab.at[ik.at[j]], kr.at[dsl], semG).wait()
            pltpu.make_async_copy(ktab.at[iv.at[j]], vr.at[dsl], semG).wait()

    def _compute_scatter(which):
        qr, kr, vr = bufs[which][5], bufs[which][6], bufs[which][7]
        dsc = bufs[which][10]

        def _grp(g, _):
            base = g * 16
            ps = [qr[base + t, :] * kr[base + t, :] for t in range(16)]
            for b in (1, 2, 4, 8):
                mask = (iota16 & b) == 0
                pidx = iota16 ^ b
                nxt = []
                for k in range(len(ps) // 2):
                    u = ps[2 * k]
                    w = ps[2 * k + 1]
                    u = u + _perm(u, pidx)
                    w = w + _perm(w, pidx)
                    nxt.append(jnp.where(mask, u, w))
                ps = nxt
            ev = jnp.exp(ps[0])
            eb[pl.ds(base, 16)] = ev
            for t in range(16):
                msg[base + t, :] = vr[base + t, :] * jnp.full((16,), ev[t])
            return 0
        lax.fori_loop(0, 32, _grp, 0)

        for j in range(4):
            ssl = pl.ds(j * 128, 128)
            pltpu.async_copy(msg.at[ssl], agg_sh.at[dsc.at[j]], semS, add=True)
            pltpu.async_copy(eb.at[ssl], den_sh.at[dsc.at[j]], semS, add=True)
        for j in range(4):
            ssl = pl.ds(j * 128, 128)
            pltpu.make_async_copy(msg.at[ssl], agg_sh.at[dsc.at[j]], semS).wait()
            pltpu.make_async_copy(eb.at[ssl], den_sh.at[dsc.at[j]], semS).wait()

    def _head(hh, _):
        h = c * 4 + hh
        def _zi(i, _):
            msg[i, :] = zrow
            return 0
        lax.fori_loop(0, C, _zi, 0)

        def _zid(i, _):
            eb[pl.ds(i * 16, 16)] = zrow
            return 0
        lax.fori_loop(0, C // 16, _zid, 0)
        for w in range(7):
            wsz = 512 if w < 6 else 64
            pltpu.sync_copy(msg.at[pl.ds(0, wsz)],
                            agg_sh.at[pl.ds(base_row + w * 512, wsz)])
            pltpu.sync_copy(eb.at[pl.ds(0, wsz)],
                            den_sh.at[pl.ds(base_row + w * 512, wsz)])
        plsc.subcore_barrier()

        _idx_load(0, 0)
        _idx_wait(0)
        _idx_compute(h, 0)
        _gather_fire(0)
        _idx_load(1, 1)

        def _pair(ii, _):
            _idx_wait(1)
            _idx_compute(h, 1)
            _gather_fire(1)
            _idx_load(2 * ii + 2, 0)
            _gather_wait(0)
            _compute_scatter(0)
            _idx_wait(0)
            _idx_compute(h, 0)
            _gather_fire(0)
            _idx_load(2 * ii + 3, 1)
            _gather_wait(1)
            _compute_scatter(1)
            return 0
        lax.fori_loop(0, CH // 2, _pair, 0)
        _gather_wait(0)
        _idx_wait(1)
        plsc.subcore_barrier()

        for w in range(7):
            wsz = 512 if w < 6 else 64
            r0 = base_row + w * 512
            pltpu.sync_copy(agg_sh.at[pl.ds(r0, wsz)], msg.at[pl.ds(0, wsz)])
            pltpu.sync_copy(den_sh.at[pl.ds(r0, wsz)], eb.at[pl.ds(0, wsz)])

            def _div(j, _):
                r16 = 1.0 / (eb[pl.ds(j * 16, 16)] + 1e-16)
                for t in range(16):
                    r = j * 16 + t
                    msg[r, :] = msg[r, :] * jnp.full((16,), r16[t])
                return 0
            lax.fori_loop(0, wsz // 16, _div, 0)
            pltpu.sync_copy(msg.at[pl.ds(0, wsz)], agg_out.at[pl.ds(r0, wsz), h])
        return 0
    lax.fori_loop(0, 4, _head, 0)


def _edge_pass(cat_dst, cat_src, si2d, di2d):
    f = pl.kernel(
        _edge_body,
        out_type=jax.ShapeDtypeStruct((NP, HEADS, DH), jnp.float32),
        mesh=_SC_MESH,
        compiler_params=pltpu.CompilerParams(use_tc_tiling_on_sc=False),
        scratch_types=[
            pltpu.VMEM((4, 128), jnp.int32),
            pltpu.VMEM((4, 128), jnp.int32),
            pltpu.VMEM((4, 128), jnp.int32),
            pltpu.VMEM((4, 128), jnp.int32),
            pltpu.VMEM((4, 128), jnp.int32),
            pltpu.VMEM((4, 128), jnp.int32),
            pltpu.VMEM((4, 128), jnp.int32),
            pltpu.VMEM((4, 128), jnp.int32),
            pltpu.VMEM((4, 128), jnp.int32),
            pltpu.VMEM((4, 128), jnp.int32),
            pltpu.VMEM((4, 128), jnp.int32),
            pltpu.VMEM((4, 128), jnp.int32),
            pltpu.VMEM((C, 16), jnp.float32),
            pltpu.VMEM((C, 16), jnp.float32),
            pltpu.VMEM((C, 16), jnp.float32),
            pltpu.VMEM((C, 16), jnp.float32),
            pltpu.VMEM((C, 16), jnp.float32),
            pltpu.VMEM((C, 16), jnp.float32),
            pltpu.VMEM((C, 16), jnp.float32),
            pltpu.VMEM((C,), jnp.float32),
            pltpu.VMEM_SHARED((NP, 16), jnp.float32),
            pltpu.VMEM_SHARED((NP,), jnp.float32),
            pltpu.SemaphoreType.DMA,
            pltpu.SemaphoreType.DMA,
            pltpu.SemaphoreType.DMA,
            pltpu.SemaphoreType.DMA,
            pltpu.SemaphoreType.DMA,
        ],
    )
    return f(cat_dst.reshape(NP * 3 * HEADS, DH),
             cat_src.reshape(NP * 3 * HEADS, DH), si2d, di2d)


def kernel(params, x_Hash, x_Address, ei_h2a, ei_a2h):
    x = {
        "Hash": jnp.pad(x_Hash, ((0, NP - N), (0, 0))),
        "Address": jnp.pad(x_Address, ((0, NP - N), (0, 0))),
    }
    eis = {}
    for name, ei in (("h2a", ei_h2a), ("a2h", ei_a2h)):
        si = jnp.pad(ei[0].astype(jnp.int32), (0, EIP - E))
        di = jnp.pad(ei[1].astype(jnp.int32), (0, EIP - E), constant_values=DUMMY)
        eis[name] = (si.reshape(-1, 128), di.reshape(-1, 128))

    h = {}
    for nt in NODE_TYPES:
        W, b = params["lin_in"][nt]
        h[nt] = _linear(x[nt], W, b, relu=True)

    for layer in params["layers"]:
        cat = {}
        for nt in NODE_TYPES:
            name = SRC_OF[nt]
            a_rel, m_rel, p_rel = layer["rel"][name]
            scale = p_rel / np.float32(np.sqrt(DH))
            A = _blockdiag(a_rel * scale[:, None, None])
            M = _blockdiag(m_rel)
            Wq, bq = layer["q"][nt]
            Wk, bk = layer["k"][nt]
            Wv, bv = layer["v"][nt]
            Wcat = jnp.concatenate([Wq, Wk @ A, Wv @ M], axis=1)
            bcat = jnp.concatenate([bq, bk @ A, bv @ M])
            cat[nt] = _linear(h[nt], Wcat, bcat)

        agg = {}
        for (src, name, dst) in EDGE_TYPES:
            si2d, di2d = eis[name]
            agg[dst] = _edge_pass(cat[dst], cat[src], si2d, di2d)

        new_h = {}
        for nt in NODE_TYPES:
            Wa, ba = layer["a"][nt]
            new_h[nt] = _out_stage(agg[nt].reshape(NP, HID), h[nt], Wa, ba,
                                   layer["skip"][nt])
        h = new_h

    W, b = params["lin"]
    out, ls = _final_stage(h["Hash"], W, b)
    return out[:N], ls[:N]
